# baseline (speedup 1.0000x reference)
import jax
import jax.numpy as jnp
from jax import lax
from jax.experimental import pallas as pl
from jax.experimental.pallas import tpu as pltpu
from jax.experimental.pallas import tpu_sc as plsc

N = 10000
E = 320000
F = 128
C = 4
EPS = 1e-5

NC = 2
NS = 16
NW = NC * NS
K = 32
NCHUNK = 320
TCHUNK = NCHUNK // 4
EPT = NCHUNK * K
EPAD = NW * EPT
RPT = 632
NP = RPT * NS
CW = 32


def _sc_agg_body(x_hbm, src_hbm, dst_hbm, zf_hbm, parts_hbm,
                 idx_src_v, idx_dst_v, rows0_v, rows1_v, rows2_v, rows3_v,
                 acc_sh, sem0, sem1, sem2, sem3):
    c = lax.axis_index("c")
    s = lax.axis_index("s")
    wid = c * NS + s
    r0 = s * RPT

    pltpu.sync_copy(zf_hbm.at[pl.ds(r0, RPT)], acc_sh.at[pl.ds(r0, RPT)])
    pltpu.sync_copy(src_hbm.at[wid], idx_src_v)
    pltpu.sync_copy(dst_hbm.at[wid], idx_dst_v)
    plsc.subcore_barrier()

    def gather(j, buf, sem):
        pltpu.async_copy(x_hbm.at[idx_src_v.at[pl.ds(j * K, K)]], buf, sem)

    def drain(buf, sem):
        pltpu.make_async_copy(x_hbm.at[idx_src_v.at[pl.ds(0, K)]], buf,
                              sem).wait()

    def scatter(j, buf):
        pltpu.sync_copy(buf, acc_sh.at[idx_dst_v.at[pl.ds(j * K, K)]],
                        add=True)

    gather(0, rows0_v, sem0)
    gather(1, rows1_v, sem1)
    gather(2, rows2_v, sem2)

    def ring(t, carry):
        a = 4 * t
        gather(a + 3, rows3_v, sem3)
        drain(rows0_v, sem0)
        scatter(a, rows0_v)
        gather(a + 4, rows0_v, sem0)
        drain(rows1_v, sem1)
        scatter(a + 1, rows1_v)
        gather(a + 5, rows1_v, sem1)
        drain(rows2_v, sem2)
        scatter(a + 2, rows2_v)
        gather(a + 6, rows2_v, sem2)
        drain(rows3_v, sem3)
        scatter(a + 3, rows3_v)
        return carry

    lax.fori_loop(0, TCHUNK - 1, ring, 0)
    a = NCHUNK - 4
    gather(a + 3, rows3_v, sem3)
    drain(rows0_v, sem0)
    scatter(a, rows0_v)
    drain(rows1_v, sem1)
    scatter(a + 1, rows1_v)
    drain(rows2_v, sem2)
    scatter(a + 2, rows2_v)
    drain(rows3_v, sem3)
    scatter(a + 3, rows3_v)
    plsc.subcore_barrier()

    pltpu.sync_copy(acc_sh.at[pl.ds(r0, RPT)], parts_hbm.at[c, pl.ds(r0, RPT)])


def _sc_cnt_body(dst_hbm, zf_hbm, ones_hbm, cnt_hbm,
                 idx_dst_v, ones_v, acc_sh, sem):
    c = lax.axis_index("c")
    s = lax.axis_index("s")
    wid = c * NS + s
    r0 = s * RPT

    pltpu.sync_copy(zf_hbm.at[pl.ds(r0, RPT)], acc_sh.at[pl.ds(r0, RPT)])
    pltpu.sync_copy(dst_hbm.at[wid], idx_dst_v)
    pltpu.sync_copy(ones_hbm, ones_v)
    plsc.subcore_barrier()

    def chunk(j, carry):
        pltpu.sync_copy(ones_v, acc_sh.at[idx_dst_v.at[pl.ds(j * K, K)]],
                        add=True)
        return carry

    lax.fori_loop(0, NCHUNK, chunk, 0)
    plsc.subcore_barrier()
    pltpu.sync_copy(acc_sh.at[pl.ds(r0, RPT)], cnt_hbm.at[c, pl.ds(r0, RPT)])


def _sc_mesh():
    return plsc.VectorSubcoreMesh(
        core_axis_name="c", subcore_axis_name="s", num_cores=NC, num_subcores=NS
    )


def _make_sc_agg():
    return pl.kernel(
        _sc_agg_body,
        out_type=[jax.ShapeDtypeStruct((NC, NP, F), jnp.float32)],
        mesh=_sc_mesh(),
        scratch_types=(
            [pltpu.VMEM((EPT,), jnp.int32),
             pltpu.VMEM((EPT,), jnp.int32)]
            + [pltpu.VMEM((K, F), jnp.float32)] * 4
            + [pltpu.VMEM_SHARED((NP, F), jnp.float32)]
            + [pltpu.SemaphoreType.DMA] * 4
        ),
    )


def _make_sc_cnt():
    return pl.kernel(
        _sc_cnt_body,
        out_type=[jax.ShapeDtypeStruct((NC, NP, F), jnp.float32)],
        mesh=_sc_mesh(),
        scratch_types=[
            pltpu.VMEM((EPT,), jnp.int32),
            pltpu.VMEM((K, F), jnp.float32),
            pltpu.VMEM_SHARED((NP, F), jnp.float32),
            pltpu.SemaphoreType.DMA,
        ],
    )


def _tc_layer1(parts_ref, cntp_ref, h_ref, wlt_ref, bl_ref, wrt_ref, g_ref,
               be_ref, out_ref, inv_ref):
    cnt = cntp_ref[0, :N, 0:1] + cntp_ref[1, :N, 0:1]
    inv = 1.0 / jnp.maximum(cnt, 1.0)
    inv_ref[...] = inv
    agg = parts_ref[0, :N] + parts_ref[1, :N]
    mean = agg * inv
    z = (jnp.dot(mean, wlt_ref[...], preferred_element_type=jnp.float32)
         + jnp.dot(h_ref[...], wrt_ref[...], preferred_element_type=jnp.float32)
         + bl_ref[...])
    mu = jnp.mean(z, axis=0, keepdims=True)
    var = jnp.mean((z - mu) ** 2, axis=0, keepdims=True)
    out_ref[...] = jnp.maximum((z - mu) / jnp.sqrt(var + EPS) * g_ref[...]
                               + be_ref[...], 0.0)


def _tc_layer2(parts_ref, inv_ref, h_ref, wlt_ref, bl_ref, wrt_ref, g_ref,
               be_ref, out_ref):
    agg = parts_ref[0, :N] + parts_ref[1, :N]
    mean = agg * inv_ref[...]
    z = (jnp.dot(mean, wlt_ref[...], preferred_element_type=jnp.float32)
         + jnp.dot(h_ref[...], wrt_ref[...], preferred_element_type=jnp.float32)
         + bl_ref[...])
    mu = jnp.mean(z, axis=0, keepdims=True)
    var = jnp.mean((z - mu) ** 2, axis=0, keepdims=True)
    out_ref[...] = jnp.maximum((z - mu) / jnp.sqrt(var + EPS) * g_ref[...]
                               + be_ref[...], 0.0)


def _tc_layer3(parts_ref, inv_ref, h_ref, wlt_ref, bl_ref, wrt_ref, g_ref,
               be_ref, wft_ref, bf_ref, wct_ref, bc_ref, out_ref):
    agg = parts_ref[0, :N] + parts_ref[1, :N]
    mean = agg * inv_ref[...]
    z = (jnp.dot(mean, wlt_ref[...], preferred_element_type=jnp.float32)
         + jnp.dot(h_ref[...], wrt_ref[...], preferred_element_type=jnp.float32)
         + bl_ref[...])
    mu = jnp.mean(z, axis=0, keepdims=True)
    var = jnp.mean((z - mu) ** 2, axis=0, keepdims=True)
    h3 = jnp.maximum((z - mu) / jnp.sqrt(var + EPS) * g_ref[...]
                     + be_ref[...], 0.0)
    hf = jnp.maximum(
        jnp.dot(h3, wft_ref[...], preferred_element_type=jnp.float32)
        + bf_ref[...], 0.0)
    out_ref[...] = (jnp.dot(hf, wct_ref[...], preferred_element_type=jnp.float32)
                    + bc_ref[...])


def kernel(x, edge_index, Wl1, bl1, Wr1, g1, be1, Wl2, bl2, Wr2, g2, be2,
           Wl3, bl3, Wr3, g3, be3, Wf, bf, Wc, bc):
    pad = EPAD - E
    pad_src = (jnp.arange(pad, dtype=jnp.int32) * 61) % N
    pad_dst = N + (jnp.arange(pad, dtype=jnp.int32) % (NP - N))
    src3 = jnp.concatenate([edge_index[0], pad_src]).reshape(NW, EPT)
    dst3 = jnp.concatenate([edge_index[1], pad_dst]).reshape(NW, EPT)
    zf = jnp.zeros((NP, F), jnp.float32)
    ones = jnp.ones((K, F), jnp.float32)

    agg = _make_sc_agg()
    cnt_kernel = _make_sc_cnt()

    row = lambda v: v.reshape(1, -1)

    cntp, = cnt_kernel(dst3, zf, ones)
    parts1, = agg(x, src3, dst3, zf)
    h1, inv = pl.pallas_call(
        _tc_layer1,
        out_shape=[jax.ShapeDtypeStruct((N, F), jnp.float32),
                   jax.ShapeDtypeStruct((N, 1), jnp.float32)],
    )(parts1, cntp, x, Wl1.T, row(bl1), Wr1.T, row(g1), row(be1))

    parts2, = agg(h1, src3, dst3, zf)
    h2, = pl.pallas_call(
        _tc_layer2,
        out_shape=[jax.ShapeDtypeStruct((N, F), jnp.float32)],
    )(parts2, inv, h1, Wl2.T, row(bl2), Wr2.T, row(g2), row(be2))

    parts3, = agg(h2, src3, dst3, zf)
    out, = pl.pallas_call(
        _tc_layer3,
        out_shape=[jax.ShapeDtypeStruct((N, C), jnp.float32)],
    )(parts3, inv, h2, Wl3.T, row(bl3), Wr3.T, row(g3), row(be3),
      Wf.T, row(bf), Wc.T, row(bc))
    return out


# speedup vs baseline: 1.1132x; 1.1132x over previous
import jax
import jax.numpy as jnp
from jax import lax
from jax.experimental import pallas as pl
from jax.experimental.pallas import tpu as pltpu
from jax.experimental.pallas import tpu_sc as plsc

N = 10000
E = 320000
F = 128
C = 4
EPS = 1e-5

NC = 2
NS = 16
NW = NC * NS
K = 56
NCHUNK = 184
TCHUNK = NCHUNK // 4
EPT = NCHUNK * K
EPAD = NW * EPT
RPT = 632
NP = RPT * NS
CW = 32


def _sc_agg_body(x_hbm, src_hbm, dst_hbm, zf_hbm, parts_hbm,
                 idx_src_v, idx_dst_v, rows0_v, rows1_v, rows2_v, rows3_v,
                 acc_sh, sem0, sem1, sem2, sem3):
    c = lax.axis_index("c")
    s = lax.axis_index("s")
    wid = c * NS + s
    r0 = s * RPT

    pltpu.sync_copy(zf_hbm.at[pl.ds(r0, RPT)], acc_sh.at[pl.ds(r0, RPT)])
    pltpu.sync_copy(src_hbm.at[wid], idx_src_v)
    pltpu.sync_copy(dst_hbm.at[wid], idx_dst_v)
    plsc.subcore_barrier()

    def gather(j, buf, sem):
        pltpu.async_copy(x_hbm.at[idx_src_v.at[pl.ds(j * K, K)]], buf, sem)

    def drain(buf, sem):
        pltpu.make_async_copy(x_hbm.at[idx_src_v.at[pl.ds(0, K)]], buf,
                              sem).wait()

    def scatter(j, buf):
        pltpu.sync_copy(buf, acc_sh.at[idx_dst_v.at[pl.ds(j * K, K)]],
                        add=True)

    gather(0, rows0_v, sem0)
    gather(1, rows1_v, sem1)
    gather(2, rows2_v, sem2)

    def ring(t, carry):
        a = 4 * t
        gather(a + 3, rows3_v, sem3)
        drain(rows0_v, sem0)
        scatter(a, rows0_v)
        gather(a + 4, rows0_v, sem0)
        drain(rows1_v, sem1)
        scatter(a + 1, rows1_v)
        gather(a + 5, rows1_v, sem1)
        drain(rows2_v, sem2)
        scatter(a + 2, rows2_v)
        gather(a + 6, rows2_v, sem2)
        drain(rows3_v, sem3)
        scatter(a + 3, rows3_v)
        return carry

    lax.fori_loop(0, TCHUNK - 1, ring, 0)
    a = NCHUNK - 4
    gather(a + 3, rows3_v, sem3)
    drain(rows0_v, sem0)
    scatter(a, rows0_v)
    drain(rows1_v, sem1)
    scatter(a + 1, rows1_v)
    drain(rows2_v, sem2)
    scatter(a + 2, rows2_v)
    drain(rows3_v, sem3)
    scatter(a + 3, rows3_v)
    plsc.subcore_barrier()

    pltpu.sync_copy(acc_sh.at[pl.ds(r0, RPT)], parts_hbm.at[c, pl.ds(r0, RPT)])


def _sc_cnt_body(dst_hbm, zf_hbm, ones_hbm, cnt_hbm,
                 idx_dst_v, ones_v, acc_sh, sem):
    c = lax.axis_index("c")
    s = lax.axis_index("s")
    wid = c * NS + s
    r0 = s * RPT

    pltpu.sync_copy(zf_hbm.at[pl.ds(r0, RPT)], acc_sh.at[pl.ds(r0, RPT)])
    pltpu.sync_copy(dst_hbm.at[wid], idx_dst_v)
    pltpu.sync_copy(ones_hbm, ones_v)
    plsc.subcore_barrier()

    def chunk(j, carry):
        pltpu.sync_copy(ones_v, acc_sh.at[idx_dst_v.at[pl.ds(j * K, K)]],
                        add=True)
        return carry

    lax.fori_loop(0, NCHUNK, chunk, 0)
    plsc.subcore_barrier()
    pltpu.sync_copy(acc_sh.at[pl.ds(r0, RPT)], cnt_hbm.at[c, pl.ds(r0, RPT)])


def _sc_mesh():
    return plsc.VectorSubcoreMesh(
        core_axis_name="c", subcore_axis_name="s", num_cores=NC, num_subcores=NS
    )


def _make_sc_agg():
    return pl.kernel(
        _sc_agg_body,
        out_type=[jax.ShapeDtypeStruct((NC, NP, F), jnp.float32)],
        mesh=_sc_mesh(),
        scratch_types=(
            [pltpu.VMEM((EPT,), jnp.int32),
             pltpu.VMEM((EPT,), jnp.int32)]
            + [pltpu.VMEM((K, F), jnp.float32)] * 4
            + [pltpu.VMEM_SHARED((NP, F), jnp.float32)]
            + [pltpu.SemaphoreType.DMA] * 4
        ),
    )


def _make_sc_cnt():
    return pl.kernel(
        _sc_cnt_body,
        out_type=[jax.ShapeDtypeStruct((NC, NP, F), jnp.float32)],
        mesh=_sc_mesh(),
        scratch_types=[
            pltpu.VMEM((EPT,), jnp.int32),
            pltpu.VMEM((K, F), jnp.float32),
            pltpu.VMEM_SHARED((NP, F), jnp.float32),
            pltpu.SemaphoreType.DMA,
        ],
    )


def _tc_layer1(parts_ref, cntp_ref, h_ref, wlt_ref, bl_ref, wrt_ref, g_ref,
               be_ref, out_ref, inv_ref):
    cnt = cntp_ref[0, :N, 0:1] + cntp_ref[1, :N, 0:1]
    inv = 1.0 / jnp.maximum(cnt, 1.0)
    inv_ref[...] = inv
    agg = parts_ref[0, :N] + parts_ref[1, :N]
    mean = agg * inv
    z = (jnp.dot(mean, wlt_ref[...], preferred_element_type=jnp.float32)
         + jnp.dot(h_ref[...], wrt_ref[...], preferred_element_type=jnp.float32)
         + bl_ref[...])
    mu = jnp.mean(z, axis=0, keepdims=True)
    var = jnp.mean((z - mu) ** 2, axis=0, keepdims=True)
    out_ref[...] = jnp.maximum((z - mu) / jnp.sqrt(var + EPS) * g_ref[...]
                               + be_ref[...], 0.0)


def _tc_layer2(parts_ref, inv_ref, h_ref, wlt_ref, bl_ref, wrt_ref, g_ref,
               be_ref, out_ref):
    agg = parts_ref[0, :N] + parts_ref[1, :N]
    mean = agg * inv_ref[...]
    z = (jnp.dot(mean, wlt_ref[...], preferred_element_type=jnp.float32)
         + jnp.dot(h_ref[...], wrt_ref[...], preferred_element_type=jnp.float32)
         + bl_ref[...])
    mu = jnp.mean(z, axis=0, keepdims=True)
    var = jnp.mean((z - mu) ** 2, axis=0, keepdims=True)
    out_ref[...] = jnp.maximum((z - mu) / jnp.sqrt(var + EPS) * g_ref[...]
                               + be_ref[...], 0.0)


def _tc_layer3(parts_ref, inv_ref, h_ref, wlt_ref, bl_ref, wrt_ref, g_ref,
               be_ref, wft_ref, bf_ref, wct_ref, bc_ref, out_ref):
    agg = parts_ref[0, :N] + parts_ref[1, :N]
    mean = agg * inv_ref[...]
    z = (jnp.dot(mean, wlt_ref[...], preferred_element_type=jnp.float32)
         + jnp.dot(h_ref[...], wrt_ref[...], preferred_element_type=jnp.float32)
         + bl_ref[...])
    mu = jnp.mean(z, axis=0, keepdims=True)
    var = jnp.mean((z - mu) ** 2, axis=0, keepdims=True)
    h3 = jnp.maximum((z - mu) / jnp.sqrt(var + EPS) * g_ref[...]
                     + be_ref[...], 0.0)
    hf = jnp.maximum(
        jnp.dot(h3, wft_ref[...], preferred_element_type=jnp.float32)
        + bf_ref[...], 0.0)
    out_ref[...] = (jnp.dot(hf, wct_ref[...], preferred_element_type=jnp.float32)
                    + bc_ref[...])


def kernel(x, edge_index, Wl1, bl1, Wr1, g1, be1, Wl2, bl2, Wr2, g2, be2,
           Wl3, bl3, Wr3, g3, be3, Wf, bf, Wc, bc):
    pad = EPAD - E
    pad_src = (jnp.arange(pad, dtype=jnp.int32) * 61) % N
    pad_dst = N + (jnp.arange(pad, dtype=jnp.int32) % (NP - N))
    src3 = jnp.concatenate([edge_index[0], pad_src]).reshape(NW, EPT)
    dst3 = jnp.concatenate([edge_index[1], pad_dst]).reshape(NW, EPT)
    zf = jnp.zeros((NP, F), jnp.float32)
    ones = jnp.ones((K, F), jnp.float32)

    agg = _make_sc_agg()
    cnt_kernel = _make_sc_cnt()

    row = lambda v: v.reshape(1, -1)

    cntp, = cnt_kernel(dst3, zf, ones)
    parts1, = agg(x, src3, dst3, zf)
    h1, inv = pl.pallas_call(
        _tc_layer1,
        out_shape=[jax.ShapeDtypeStruct((N, F), jnp.float32),
                   jax.ShapeDtypeStruct((N, 1), jnp.float32)],
    )(parts1, cntp, x, Wl1.T, row(bl1), Wr1.T, row(g1), row(be1))

    parts2, = agg(h1, src3, dst3, zf)
    h2, = pl.pallas_call(
        _tc_layer2,
        out_shape=[jax.ShapeDtypeStruct((N, F), jnp.float32)],
    )(parts2, inv, h1, Wl2.T, row(bl2), Wr2.T, row(g2), row(be2))

    parts3, = agg(h2, src3, dst3, zf)
    out, = pl.pallas_call(
        _tc_layer3,
        out_shape=[jax.ShapeDtypeStruct((N, C), jnp.float32)],
    )(parts3, inv, h2, Wl3.T, row(bl3), Wr3.T, row(g3), row(be3),
      Wf.T, row(bf), Wc.T, row(bc))
    return out
